# Initial kernel scaffold; baseline (speedup 1.0000x reference)
import functools

import jax
import jax.numpy as jnp
from jax import lax
from jax.experimental import pallas as pl
from jax.experimental.pallas import tpu as pltpu
from jax.experimental.pallas import tpu_sc as plsc

N_NODES = 100000
N_GRAPHS = 512
D_NODE = 128
D_EDGE = 16
D_GLOBAL = 64
HIDDEN = 128

_NC = 2
_NS = 16
_NW = _NC * _NS

_CHUNK = 128
_NFULL = N_NODES // _CHUNK
_TAIL = N_NODES - _NFULL * _CHUNK
_TAIL_BASE = _NFULL * _CHUNK
_ACC_ROWS = N_GRAPHS + 8

_mesh = plsc.VectorSubcoreMesh(core_axis_name="c", subcore_axis_name="s")


@functools.partial(
    pl.kernel,
    mesh=_mesh,
    out_type=(
        jax.ShapeDtypeStruct((_NC, N_GRAPHS, D_NODE), jnp.float32),
        jax.ShapeDtypeStruct((_NC, N_GRAPHS, D_EDGE), jnp.float32),
        jax.ShapeDtypeStruct((_NC, N_GRAPHS, 16), jnp.float32),
    ),
    scratch_types=(
        pltpu.VMEM((_CHUNK, D_NODE), jnp.float32),
        pltpu.VMEM((_CHUNK, D_EDGE), jnp.float32),
        pltpu.VMEM((_CHUNK, 16), jnp.float32),
        pltpu.VMEM((_CHUNK,), jnp.int32),
        pltpu.VMEM_SHARED((_ACC_ROWS, D_NODE), jnp.float32),
        pltpu.VMEM_SHARED((_ACC_ROWS, D_EDGE), jnp.float32),
        pltpu.VMEM_SHARED((_ACC_ROWS, 16), jnp.float32),
    ),
)
def _segment_sums(node_hbm, edge_hbm, batch_hbm, zn_hbm, ze_hbm, zc_hbm,
                  ones_hbm, out_n, out_e, out_c,
                  node_v, edge_v, ones_v, idx_v, accn_s, acce_s, accc_s):
    cid = lax.axis_index("c")
    sid = lax.axis_index("s")
    wid = sid * _NC + cid

    @pl.when(sid == 0)
    def _init():
        pltpu.sync_copy(zn_hbm, accn_s.at[pl.ds(0, N_GRAPHS)])
        pltpu.sync_copy(ze_hbm, acce_s.at[pl.ds(0, N_GRAPHS)])
        pltpu.sync_copy(zc_hbm, accc_s.at[pl.ds(0, N_GRAPHS)])

    pltpu.sync_copy(ones_hbm, ones_v)
    plsc.subcore_barrier()

    n_chunks = _NFULL // _NW + jnp.where(wid < _NFULL % _NW, 1, 0)

    def _body(i, carry):
        base = (wid + i * _NW) * _CHUNK
        pltpu.sync_copy(batch_hbm.at[pl.ds(base, _CHUNK)], idx_v)
        pltpu.sync_copy(node_hbm.at[pl.ds(base, _CHUNK)], node_v)
        pltpu.sync_copy(edge_hbm.at[pl.ds(base, _CHUNK)], edge_v)
        pltpu.sync_copy(node_v, accn_s.at[idx_v], add=True)
        pltpu.sync_copy(edge_v, acce_s.at[idx_v], add=True)
        pltpu.sync_copy(ones_v, accc_s.at[idx_v], add=True)
        return carry

    lax.fori_loop(0, n_chunks, _body, 0)

    @pl.when(wid == _NW - 1)
    def _tail():
        for j in range(_TAIL // 16, _CHUNK // 16):
            idx_v[pl.ds(j * 16, 16)] = jnp.full((16,), N_GRAPHS, jnp.int32)
        pltpu.sync_copy(batch_hbm.at[pl.ds(_TAIL_BASE, _TAIL)],
                        idx_v.at[pl.ds(0, _TAIL)])
        pltpu.sync_copy(node_hbm.at[pl.ds(_TAIL_BASE, _TAIL)],
                        node_v.at[pl.ds(0, _TAIL)])
        pltpu.sync_copy(edge_hbm.at[pl.ds(_TAIL_BASE, _TAIL)],
                        edge_v.at[pl.ds(0, _TAIL)])
        pltpu.sync_copy(node_v, accn_s.at[idx_v], add=True)
        pltpu.sync_copy(edge_v, acce_s.at[idx_v], add=True)
        pltpu.sync_copy(ones_v, accc_s.at[idx_v], add=True)

    plsc.subcore_barrier()

    @pl.when(sid == 0)
    def _flush():
        pltpu.sync_copy(accn_s.at[pl.ds(0, N_GRAPHS)], out_n.at[cid])
        pltpu.sync_copy(acce_s.at[pl.ds(0, N_GRAPHS)], out_e.at[cid])
        pltpu.sync_copy(accc_s.at[pl.ds(0, N_GRAPHS)], out_c.at[cid])


def _mlp_body(nsum_ref, esum_ref, csum_ref, u_ref, w1n_ref, w1e_ref,
              w1u_ref, b1_ref, w2_ref, b2_ref, out_ref):
    cnt = csum_ref[0, :, 0:1] + csum_ref[1, :, 0:1]
    cnt = jnp.maximum(cnt, 1.0)
    nbar = (nsum_ref[0] + nsum_ref[1]) / cnt
    ebar = (esum_ref[0] + esum_ref[1]) / cnt
    hp = jnp.float32
    h = (jnp.dot(nbar, w1n_ref[...], preferred_element_type=hp,
                 precision=lax.Precision.HIGHEST)
         + jnp.dot(ebar, w1e_ref[...], preferred_element_type=hp,
                   precision=lax.Precision.HIGHEST)
         + jnp.dot(u_ref[...], w1u_ref[...], preferred_element_type=hp,
                   precision=lax.Precision.HIGHEST)
         + b1_ref[...][None, :])
    h = jnp.maximum(h, 0.0)
    y = jnp.dot(h, w2_ref[...], preferred_element_type=hp,
                precision=lax.Precision.HIGHEST) + b2_ref[...][None, :]
    out_ref[...] = jnp.maximum(y, 0.0)


_mlp_call = pl.pallas_call(
    _mlp_body,
    out_shape=jax.ShapeDtypeStruct((N_GRAPHS, D_GLOBAL), jnp.float32),
)


def kernel(node_attr_prime, edge_out_bar, u, batch, W1, b1, W2, b2):
    batch = batch.astype(jnp.int32)
    zn = jnp.zeros((N_GRAPHS, D_NODE), jnp.float32)
    ze = jnp.zeros((N_GRAPHS, D_EDGE), jnp.float32)
    zc = jnp.zeros((N_GRAPHS, 16), jnp.float32)
    ones = jnp.ones((_CHUNK, 16), jnp.float32)
    nsum, esum, csum = _segment_sums(node_attr_prime, edge_out_bar, batch,
                                     zn, ze, zc, ones)
    return _mlp_call(nsum, esum, csum, u,
                     W1[:D_NODE], W1[D_NODE:D_NODE + D_EDGE],
                     W1[D_NODE + D_EDGE:], b1, W2, b2)


# speedup vs baseline: 4.1568x; 4.1568x over previous
import functools

import jax
import jax.numpy as jnp
from jax import lax
from jax.experimental import pallas as pl
from jax.experimental.pallas import tpu as pltpu
from jax.experimental.pallas import tpu_sc as plsc

N_NODES = 100000
N_GRAPHS = 512
D_NODE = 128
D_EDGE = 16
D_GLOBAL = 64
HIDDEN = 128

_NC = 2
_NS = 16
_NW = _NC * _NS

_CHUNK = 128
_NFULL = N_NODES // _CHUNK
_TAIL = N_NODES - _NFULL * _CHUNK
_TAIL_BASE = _NFULL * _CHUNK
_ACC_ROWS = N_GRAPHS + 8

_EP = D_NODE // D_EDGE
_PCHUNK = _CHUNK // _EP
_PTAIL = _TAIL // _EP

_mesh = plsc.VectorSubcoreMesh(core_axis_name="c", subcore_axis_name="s")


@functools.partial(
    pl.kernel,
    mesh=_mesh,
    out_type=(
        jax.ShapeDtypeStruct((_NC, N_GRAPHS, D_NODE), jnp.float32),
        jax.ShapeDtypeStruct((_NC, N_GRAPHS, D_NODE), jnp.float32),
    ),
    scratch_types=(
        pltpu.VMEM((_CHUNK, D_NODE), jnp.float32),
        pltpu.VMEM((_CHUNK, D_NODE), jnp.float32),
        pltpu.VMEM((_PCHUNK, D_NODE), jnp.float32),
        pltpu.VMEM((_CHUNK,), jnp.int32),
        pltpu.VMEM_SHARED((_ACC_ROWS, D_NODE), jnp.float32),
        pltpu.VMEM_SHARED((_ACC_ROWS, D_NODE), jnp.float32),
    ),
)
def _segment_sums(node_hbm, edge2_hbm, batch_hbm, zeros_hbm, ones_hbm,
                  out_n, out_ec,
                  node_v, ec_v, estage_v, idx_v, accn_s, accec_s):
    cid = lax.axis_index("c")
    sid = lax.axis_index("s")
    wid = sid * _NC + cid

    @pl.when(sid == 0)
    def _init():
        pltpu.sync_copy(zeros_hbm, accn_s.at[pl.ds(0, N_GRAPHS)])
        pltpu.sync_copy(zeros_hbm, accec_s.at[pl.ds(0, N_GRAPHS)])

    pltpu.sync_copy(ones_hbm, ec_v)
    plsc.subcore_barrier()

    def _unpack_edges(n_pseudo):
        for p in range(n_pseudo):
            for s in range(_EP):
                ec_v[p * _EP + s, pl.ds(0, D_EDGE)] = (
                    estage_v[p, pl.ds(s * D_EDGE, D_EDGE)])

    n_chunks = _NFULL // _NW + jnp.where(wid < _NFULL % _NW, 1, 0)

    def _body(i, carry):
        c = wid + i * _NW
        base = c * _CHUNK
        pltpu.sync_copy(batch_hbm.at[pl.ds(base, _CHUNK)], idx_v)
        pltpu.sync_copy(node_hbm.at[pl.ds(base, _CHUNK)], node_v)
        pltpu.sync_copy(edge2_hbm.at[pl.ds(c * _PCHUNK, _PCHUNK)], estage_v)
        _unpack_edges(_PCHUNK)
        pltpu.sync_copy(node_v, accn_s.at[idx_v], add=True)
        pltpu.sync_copy(ec_v, accec_s.at[idx_v], add=True)
        return carry

    lax.fori_loop(0, n_chunks, _body, 0)

    @pl.when(wid == _NW - 1)
    def _tail():
        for j in range(_TAIL // 16, _CHUNK // 16):
            idx_v[pl.ds(j * 16, 16)] = jnp.full((16,), N_GRAPHS, jnp.int32)
        pltpu.sync_copy(batch_hbm.at[pl.ds(_TAIL_BASE, _TAIL)],
                        idx_v.at[pl.ds(0, _TAIL)])
        pltpu.sync_copy(node_hbm.at[pl.ds(_TAIL_BASE, _TAIL)],
                        node_v.at[pl.ds(0, _TAIL)])
        pltpu.sync_copy(edge2_hbm.at[pl.ds(_TAIL_BASE // _EP, _PTAIL)],
                        estage_v.at[pl.ds(0, _PTAIL)])
        _unpack_edges(_PTAIL)
        pltpu.sync_copy(node_v, accn_s.at[idx_v], add=True)
        pltpu.sync_copy(ec_v, accec_s.at[idx_v], add=True)

    plsc.subcore_barrier()

    @pl.when(sid == 0)
    def _flush():
        pltpu.sync_copy(accn_s.at[pl.ds(0, N_GRAPHS)], out_n.at[cid])
        pltpu.sync_copy(accec_s.at[pl.ds(0, N_GRAPHS)], out_ec.at[cid])


def _mlp_body(nsum_ref, ecsum_ref, u_ref, w1n_ref, w1e_ref,
              w1u_ref, b1_ref, w2_ref, b2_ref, out_ref):
    ec = ecsum_ref[0] + ecsum_ref[1]
    cnt = jnp.maximum(ec[:, D_EDGE:D_EDGE + 1], 1.0)
    nbar = (nsum_ref[0] + nsum_ref[1]) / cnt
    ebar = ec[:, :D_EDGE] / cnt
    hp = jnp.float32
    h = (jnp.dot(nbar, w1n_ref[...], preferred_element_type=hp,
                 precision=lax.Precision.HIGHEST)
         + jnp.dot(ebar, w1e_ref[...], preferred_element_type=hp,
                   precision=lax.Precision.HIGHEST)
         + jnp.dot(u_ref[...], w1u_ref[...], preferred_element_type=hp,
                   precision=lax.Precision.HIGHEST)
         + b1_ref[...][None, :])
    h = jnp.maximum(h, 0.0)
    y = jnp.dot(h, w2_ref[...], preferred_element_type=hp,
                precision=lax.Precision.HIGHEST) + b2_ref[...][None, :]
    out_ref[...] = jnp.maximum(y, 0.0)


_mlp_call = pl.pallas_call(
    _mlp_body,
    out_shape=jax.ShapeDtypeStruct((N_GRAPHS, D_GLOBAL), jnp.float32),
)


def kernel(node_attr_prime, edge_out_bar, u, batch, W1, b1, W2, b2):
    batch = batch.astype(jnp.int32)
    edge2 = edge_out_bar.reshape(N_NODES // _EP, D_NODE)
    zeros = jnp.zeros((N_GRAPHS, D_NODE), jnp.float32)
    ones = jnp.ones((_CHUNK, D_NODE), jnp.float32)
    nsum, ecsum = _segment_sums(node_attr_prime, edge2, batch, zeros, ones)
    return _mlp_call(nsum, ecsum, u,
                     W1[:D_NODE], W1[D_NODE:D_NODE + D_EDGE],
                     W1[D_NODE + D_EDGE:], b1, W2, b2)


# speedup vs baseline: 5.6405x; 1.3569x over previous
import functools

import jax
import jax.numpy as jnp
from jax import lax
from jax.experimental import pallas as pl
from jax.experimental.pallas import tpu as pltpu
from jax.experimental.pallas import tpu_sc as plsc

N_NODES = 100000
N_GRAPHS = 512
D_NODE = 128
D_EDGE = 16
D_GLOBAL = 64
HIDDEN = 128

_NC = 2
_NS = 16
_NW = _NC * _NS

_CHUNK = 128
_NFULL = N_NODES // _CHUNK
_TAIL = N_NODES - _NFULL * _CHUNK
_TAIL_BASE = _NFULL * _CHUNK
_ACC_ROWS = N_GRAPHS + 8

_EP = D_NODE // D_EDGE
_PCHUNK = _CHUNK // _EP
_PTAIL = _TAIL // _EP

_mesh = plsc.VectorSubcoreMesh(core_axis_name="c", subcore_axis_name="s")


@functools.partial(
    pl.kernel,
    mesh=_mesh,
    out_type=(
        jax.ShapeDtypeStruct((_NC, N_GRAPHS, D_NODE), jnp.float32),
        jax.ShapeDtypeStruct((_NC, N_GRAPHS, D_NODE), jnp.float32),
    ),
    scratch_types=(
        pltpu.VMEM((2, _CHUNK, D_NODE), jnp.float32),
        pltpu.VMEM((_CHUNK, D_NODE), jnp.float32),
        pltpu.VMEM((2, _PCHUNK, D_NODE), jnp.float32),
        pltpu.VMEM((2, _CHUNK), jnp.int32),
        pltpu.VMEM_SHARED((_ACC_ROWS, D_NODE), jnp.float32),
        pltpu.VMEM_SHARED((_ACC_ROWS, D_NODE), jnp.float32),
        pltpu.SemaphoreType.DMA,
        pltpu.SemaphoreType.DMA,
    ),
)
def _segment_sums(node_hbm, edge2_hbm, batch_hbm, zeros_hbm, ones_hbm,
                  out_n, out_ec,
                  node_v, ec_v, estage_v, idx_v, accn_s, accec_s,
                  sem0, sem1):
    cid = lax.axis_index("c")
    sid = lax.axis_index("s")
    wid = sid * _NC + cid
    sems = (sem0, sem1)

    @pl.when(sid == 0)
    def _init():
        pltpu.sync_copy(zeros_hbm, accn_s.at[pl.ds(0, N_GRAPHS)])
        pltpu.sync_copy(zeros_hbm, accec_s.at[pl.ds(0, N_GRAPHS)])

    pltpu.sync_copy(ones_hbm, ec_v)
    plsc.subcore_barrier()

    def _unpack_edges(b, n_pseudo):
        for p in range(n_pseudo):
            for s in range(_EP):
                ec_v[p * _EP + s, pl.ds(0, D_EDGE)] = (
                    estage_v[b, p, pl.ds(s * D_EDGE, D_EDGE)])

    def _chunk(j):
        return jnp.minimum(wid + j * _NW, _NFULL - 1)

    def _start_loads(j, b):
        c = _chunk(j)
        base = c * _CHUNK
        pltpu.async_copy(batch_hbm.at[pl.ds(base, _CHUNK)], idx_v.at[b],
                         sems[b])
        pltpu.async_copy(node_hbm.at[pl.ds(base, _CHUNK)], node_v.at[b],
                         sems[b])
        pltpu.async_copy(edge2_hbm.at[pl.ds(c * _PCHUNK, _PCHUNK)],
                         estage_v.at[b], sems[b])

    def _wait_loads(j, b):
        c = _chunk(j)
        base = c * _CHUNK
        pltpu.make_async_copy(batch_hbm.at[pl.ds(base, _CHUNK)],
                              idx_v.at[b], sems[b]).wait()
        pltpu.make_async_copy(node_hbm.at[pl.ds(base, _CHUNK)],
                              node_v.at[b], sems[b]).wait()
        pltpu.make_async_copy(edge2_hbm.at[pl.ds(c * _PCHUNK, _PCHUNK)],
                              estage_v.at[b], sems[b]).wait()

    def _scatter(b):
        _unpack_edges(b, _PCHUNK)
        pltpu.sync_copy(node_v.at[b], accn_s.at[idx_v.at[b]], add=True)
        pltpu.sync_copy(ec_v, accec_s.at[idx_v.at[b]], add=True)

    _start_loads(0, 0)
    _start_loads(1, 1)

    def _body(it, carry):
        j0 = it * 2
        _wait_loads(j0, 0)
        _scatter(0)
        _start_loads(j0 + 2, 0)
        _wait_loads(j0 + 1, 1)
        _scatter(1)
        _start_loads(j0 + 3, 1)
        return carry

    _even = (_NFULL // _NW) // 2
    lax.fori_loop(0, _even, _body, 0)

    _wait_loads(2 * _even, 0)
    _wait_loads(2 * _even + 1, 1)

    @pl.when(wid < _NFULL % _NW)
    def _extra():
        _scatter(0)

    @pl.when(wid == _NW - 1)
    def _tail():
        for j in range(_TAIL // 16, _CHUNK // 16):
            idx_v[1, pl.ds(j * 16, 16)] = jnp.full((16,), N_GRAPHS, jnp.int32)
        pltpu.sync_copy(batch_hbm.at[pl.ds(_TAIL_BASE, _TAIL)],
                        idx_v.at[1, pl.ds(0, _TAIL)])
        pltpu.sync_copy(node_hbm.at[pl.ds(_TAIL_BASE, _TAIL)],
                        node_v.at[1, pl.ds(0, _TAIL)])
        pltpu.sync_copy(edge2_hbm.at[pl.ds(_TAIL_BASE // _EP, _PTAIL)],
                        estage_v.at[1, pl.ds(0, _PTAIL)])
        _unpack_edges(1, _PTAIL)
        pltpu.sync_copy(node_v.at[1], accn_s.at[idx_v.at[1]], add=True)
        pltpu.sync_copy(ec_v, accec_s.at[idx_v.at[1]], add=True)

    plsc.subcore_barrier()

    @pl.when(sid == 0)
    def _flush():
        pltpu.sync_copy(accn_s.at[pl.ds(0, N_GRAPHS)], out_n.at[cid])
        pltpu.sync_copy(accec_s.at[pl.ds(0, N_GRAPHS)], out_ec.at[cid])


def _mlp_body(nsum_ref, ecsum_ref, u_ref, w1n_ref, w1e_ref,
              w1u_ref, b1_ref, w2_ref, b2_ref, out_ref):
    ec = ecsum_ref[0] + ecsum_ref[1]
    cnt = jnp.maximum(ec[:, D_EDGE:D_EDGE + 1], 1.0)
    nbar = (nsum_ref[0] + nsum_ref[1]) / cnt
    ebar = ec[:, :D_EDGE] / cnt
    hp = jnp.float32
    h = (jnp.dot(nbar, w1n_ref[...], preferred_element_type=hp,
                 precision=lax.Precision.HIGHEST)
         + jnp.dot(ebar, w1e_ref[...], preferred_element_type=hp,
                   precision=lax.Precision.HIGHEST)
         + jnp.dot(u_ref[...], w1u_ref[...], preferred_element_type=hp,
                   precision=lax.Precision.HIGHEST)
         + b1_ref[...][None, :])
    h = jnp.maximum(h, 0.0)
    y = jnp.dot(h, w2_ref[...], preferred_element_type=hp,
                precision=lax.Precision.HIGHEST) + b2_ref[...][None, :]
    out_ref[...] = jnp.maximum(y, 0.0)


_mlp_call = pl.pallas_call(
    _mlp_body,
    out_shape=jax.ShapeDtypeStruct((N_GRAPHS, D_GLOBAL), jnp.float32),
)


def kernel(node_attr_prime, edge_out_bar, u, batch, W1, b1, W2, b2):
    batch = batch.astype(jnp.int32)
    edge2 = edge_out_bar.reshape(N_NODES // _EP, D_NODE)
    zeros = jnp.zeros((N_GRAPHS, D_NODE), jnp.float32)
    ones = jnp.ones((_CHUNK, D_NODE), jnp.float32)
    nsum, ecsum = _segment_sums(node_attr_prime, edge2, batch, zeros, ones)
    return _mlp_call(nsum, ecsum, u,
                     W1[:D_NODE], W1[D_NODE:D_NODE + D_EDGE],
                     W1[D_NODE + D_EDGE:], b1, W2, b2)


# speedup vs baseline: 5.9587x; 1.0564x over previous
import functools

import jax
import jax.numpy as jnp
from jax import lax
from jax.experimental import pallas as pl
from jax.experimental.pallas import tpu as pltpu
from jax.experimental.pallas import tpu_sc as plsc

N_NODES = 100000
N_GRAPHS = 512
D_NODE = 128
D_EDGE = 16
D_GLOBAL = 64
HIDDEN = 128

_NC = 2
_NS = 16
_NW = _NC * _NS

_CHUNK = 128
_NFULL = N_NODES // _CHUNK
_TAIL = N_NODES - _NFULL * _CHUNK
_TAIL_BASE = _NFULL * _CHUNK
_ACC_ROWS = N_GRAPHS + 8

_EP = D_NODE // D_EDGE
_PCHUNK = _CHUNK // _EP
_PTAIL = _TAIL // _EP

_mesh = plsc.VectorSubcoreMesh(core_axis_name="c", subcore_axis_name="s")


@functools.partial(
    pl.kernel,
    mesh=_mesh,
    out_type=(
        jax.ShapeDtypeStruct((_NC, N_GRAPHS, D_NODE), jnp.float32),
        jax.ShapeDtypeStruct((_NC, N_GRAPHS, D_NODE), jnp.float32),
    ),
    scratch_types=(
        pltpu.VMEM((2, _CHUNK, D_NODE), jnp.float32),
        pltpu.VMEM((_CHUNK, D_NODE), jnp.float32),
        pltpu.VMEM((2, _CHUNK, D_EDGE), jnp.float32),
        pltpu.VMEM((2, _CHUNK), jnp.int32),
        pltpu.VMEM_SHARED((_ACC_ROWS, D_NODE), jnp.float32),
        pltpu.VMEM_SHARED((_ACC_ROWS, D_NODE), jnp.float32),
        pltpu.SemaphoreType.DMA,
        pltpu.SemaphoreType.DMA,
    ),
)
def _segment_sums(node_hbm, edge_hbm, batch_hbm, zeros_hbm, ones_hbm,
                  out_n, out_ec,
                  node_v, ec_v, edge_v, idx_v, accn_s, accec_s,
                  sem0, sem1):
    cid = lax.axis_index("c")
    sid = lax.axis_index("s")
    wid = sid * _NC + cid
    sems = (sem0, sem1)

    @pl.when(sid == 0)
    def _init():
        pltpu.sync_copy(zeros_hbm, accn_s.at[pl.ds(0, N_GRAPHS)])
        pltpu.sync_copy(zeros_hbm, accec_s.at[pl.ds(0, N_GRAPHS)])

    pltpu.sync_copy(ones_hbm, ec_v)
    plsc.subcore_barrier()

    def _unpack_edges(b, n_rows):
        for r in range(n_rows):
            ec_v[r, pl.ds(0, D_EDGE)] = edge_v[b, r, :]

    def _chunk(j):
        return jnp.minimum(wid + j * _NW, _NFULL - 1)

    def _start_loads(j, b):
        c = _chunk(j)
        base = c * _CHUNK
        pltpu.async_copy(batch_hbm.at[pl.ds(base, _CHUNK)], idx_v.at[b],
                         sems[b])
        pltpu.async_copy(node_hbm.at[pl.ds(base, _CHUNK)], node_v.at[b],
                         sems[b])
        pltpu.async_copy(edge_hbm.at[pl.ds(base, _CHUNK)],
                         edge_v.at[b], sems[b])

    def _wait_loads(j, b):
        c = _chunk(j)
        base = c * _CHUNK
        pltpu.make_async_copy(batch_hbm.at[pl.ds(base, _CHUNK)],
                              idx_v.at[b], sems[b]).wait()
        pltpu.make_async_copy(node_hbm.at[pl.ds(base, _CHUNK)],
                              node_v.at[b], sems[b]).wait()
        pltpu.make_async_copy(edge_hbm.at[pl.ds(base, _CHUNK)],
                              edge_v.at[b], sems[b]).wait()

    def _scatter(b):
        _unpack_edges(b, _CHUNK)
        pltpu.sync_copy(node_v.at[b], accn_s.at[idx_v.at[b]], add=True)
        pltpu.sync_copy(ec_v, accec_s.at[idx_v.at[b]], add=True)

    _start_loads(0, 0)
    _start_loads(1, 1)

    def _body(it, carry):
        j0 = it * 2
        _wait_loads(j0, 0)
        _scatter(0)
        _start_loads(j0 + 2, 0)
        _wait_loads(j0 + 1, 1)
        _scatter(1)
        _start_loads(j0 + 3, 1)
        return carry

    _even = (_NFULL // _NW) // 2
    lax.fori_loop(0, _even, _body, 0)

    _wait_loads(2 * _even, 0)
    _wait_loads(2 * _even + 1, 1)

    @pl.when(wid < _NFULL % _NW)
    def _extra():
        _scatter(0)

    @pl.when(wid == _NW - 1)
    def _tail():
        for j in range(_TAIL // 16, _CHUNK // 16):
            idx_v[1, pl.ds(j * 16, 16)] = jnp.full((16,), N_GRAPHS, jnp.int32)
        pltpu.sync_copy(batch_hbm.at[pl.ds(_TAIL_BASE, _TAIL)],
                        idx_v.at[1, pl.ds(0, _TAIL)])
        pltpu.sync_copy(node_hbm.at[pl.ds(_TAIL_BASE, _TAIL)],
                        node_v.at[1, pl.ds(0, _TAIL)])
        pltpu.sync_copy(edge_hbm.at[pl.ds(_TAIL_BASE, _TAIL)],
                        edge_v.at[1, pl.ds(0, _TAIL)])
        _unpack_edges(1, _TAIL)
        pltpu.sync_copy(node_v.at[1], accn_s.at[idx_v.at[1]], add=True)
        pltpu.sync_copy(ec_v, accec_s.at[idx_v.at[1]], add=True)

    plsc.subcore_barrier()

    @pl.when(sid == 0)
    def _flush():
        pltpu.sync_copy(accn_s.at[pl.ds(0, N_GRAPHS)], out_n.at[cid])
        pltpu.sync_copy(accec_s.at[pl.ds(0, N_GRAPHS)], out_ec.at[cid])


def _mlp_body(nsum_ref, ecsum_ref, u_ref, w1n_ref, w1e_ref,
              w1u_ref, b1_ref, w2_ref, b2_ref, out_ref):
    ec = ecsum_ref[0] + ecsum_ref[1]
    cnt = jnp.maximum(ec[:, D_EDGE:D_EDGE + 1], 1.0)
    nbar = (nsum_ref[0] + nsum_ref[1]) / cnt
    ebar = ec[:, :D_EDGE] / cnt
    hp = jnp.float32
    h = (jnp.dot(nbar, w1n_ref[...], preferred_element_type=hp,
                 precision=lax.Precision.HIGHEST)
         + jnp.dot(ebar, w1e_ref[...], preferred_element_type=hp,
                   precision=lax.Precision.HIGHEST)
         + jnp.dot(u_ref[...], w1u_ref[...], preferred_element_type=hp,
                   precision=lax.Precision.HIGHEST)
         + b1_ref[...][None, :])
    h = jnp.maximum(h, 0.0)
    y = jnp.dot(h, w2_ref[...], preferred_element_type=hp,
                precision=lax.Precision.HIGHEST) + b2_ref[...][None, :]
    out_ref[...] = jnp.maximum(y, 0.0)


_mlp_call = pl.pallas_call(
    _mlp_body,
    out_shape=jax.ShapeDtypeStruct((N_GRAPHS, D_GLOBAL), jnp.float32),
)


def kernel(node_attr_prime, edge_out_bar, u, batch, W1, b1, W2, b2):
    batch = batch.astype(jnp.int32)
    zeros = jnp.zeros((N_GRAPHS, D_NODE), jnp.float32)
    ones = jnp.ones((_CHUNK, D_NODE), jnp.float32)
    nsum, ecsum = _segment_sums(node_attr_prime, edge_out_bar, batch,
                                zeros, ones)
    return _mlp_call(nsum, ecsum, u,
                     W1[:D_NODE], W1[D_NODE:D_NODE + D_EDGE],
                     W1[D_NODE + D_EDGE:], b1, W2, b2)


# speedup vs baseline: 5.9944x; 1.0060x over previous
import functools

import jax
import jax.numpy as jnp
from jax import lax
from jax.experimental import pallas as pl
from jax.experimental.pallas import tpu as pltpu
from jax.experimental.pallas import tpu_sc as plsc

N_NODES = 100000
N_GRAPHS = 512
D_NODE = 128
D_EDGE = 16
D_GLOBAL = 64
HIDDEN = 128

_NC = 2
_NS = 16
_NW = _NC * _NS

_CHUNK = 128
_NFULL = N_NODES // _CHUNK
_TAIL = N_NODES - _NFULL * _CHUNK
_TAIL_BASE = _NFULL * _CHUNK
_ACC_ROWS = N_GRAPHS + 8

_EP = D_NODE // D_EDGE
_PCHUNK = _CHUNK // _EP
_PTAIL = _TAIL // _EP

_mesh = plsc.VectorSubcoreMesh(core_axis_name="c", subcore_axis_name="s")


@functools.partial(
    pl.kernel,
    mesh=_mesh,
    out_type=(
        jax.ShapeDtypeStruct((_NC, N_GRAPHS, D_NODE), jnp.float32),
        jax.ShapeDtypeStruct((_NC, N_GRAPHS, D_NODE), jnp.float32),
    ),
    scratch_types=(
        pltpu.VMEM((2, _CHUNK, D_NODE), jnp.float32),
        pltpu.VMEM((_CHUNK, D_NODE), jnp.float32),
        pltpu.VMEM((2, _CHUNK, D_EDGE), jnp.float32),
        pltpu.VMEM((2, _CHUNK), jnp.int32),
        pltpu.VMEM_SHARED((_ACC_ROWS, D_NODE), jnp.float32),
        pltpu.VMEM_SHARED((_ACC_ROWS, D_NODE), jnp.float32),
        pltpu.SemaphoreType.DMA,
        pltpu.SemaphoreType.DMA,
        pltpu.SemaphoreType.DMA,
    ),
)
def _segment_sums(node_hbm, edge_hbm, batch_hbm, zeros_hbm,
                  ones_hbm, out_n, out_ec,
                  node_v, ec_v, edge_v, idx_v,
                  accn_s, accec_s, sem0, sem1, sem2):
    cid = lax.axis_index("c")
    sid = lax.axis_index("s")
    wid = sid * _NC + cid
    sems = (sem0, sem1)

    @pl.when(sid == 0)
    def _init():
        pltpu.sync_copy(zeros_hbm, accn_s.at[pl.ds(0, N_GRAPHS)])
        pltpu.sync_copy(zeros_hbm, accec_s.at[pl.ds(0, N_GRAPHS)])

    pltpu.sync_copy(ones_hbm, ec_v)
    plsc.subcore_barrier()

    def _unpack_edges(b, n_rows):
        for r in range(n_rows):
            ec_v[r, pl.ds(0, D_EDGE)] = edge_v[b, r, :]

    def _chunk(j):
        return jnp.minimum(wid + j * _NW, _NFULL - 1)

    def _start_loads(j, b):
        c = _chunk(j)
        base = c * _CHUNK
        pltpu.async_copy(batch_hbm.at[pl.ds(base, _CHUNK)], idx_v.at[b],
                         sems[b])
        pltpu.async_copy(node_hbm.at[pl.ds(base, _CHUNK)], node_v.at[b],
                         sems[b])
        pltpu.async_copy(edge_hbm.at[pl.ds(base, _CHUNK)],
                         edge_v.at[b], sems[b])

    def _wait_loads(j, b):
        c = _chunk(j)
        base = c * _CHUNK
        pltpu.make_async_copy(batch_hbm.at[pl.ds(base, _CHUNK)],
                              idx_v.at[b], sems[b]).wait()
        pltpu.make_async_copy(node_hbm.at[pl.ds(base, _CHUNK)],
                              node_v.at[b], sems[b]).wait()
        pltpu.make_async_copy(edge_hbm.at[pl.ds(base, _CHUNK)],
                              edge_v.at[b], sems[b]).wait()

    def _scatter(b):
        h = pltpu.async_copy(node_v.at[b], accn_s.at[idx_v.at[b]], sem2,
                             add=True)
        _unpack_edges(b, _CHUNK)
        pltpu.sync_copy(ec_v, accec_s.at[idx_v.at[b]], add=True)
        h.wait()

    _start_loads(0, 0)
    _start_loads(1, 1)

    def _body(it, carry):
        j0 = it * 2
        _wait_loads(j0, 0)
        _scatter(0)
        _start_loads(j0 + 2, 0)
        _wait_loads(j0 + 1, 1)
        _scatter(1)
        _start_loads(j0 + 3, 1)
        return carry

    _even = (_NFULL // _NW) // 2
    lax.fori_loop(0, _even, _body, 0)

    _wait_loads(2 * _even, 0)
    _wait_loads(2 * _even + 1, 1)

    @pl.when(wid < _NFULL % _NW)
    def _extra():
        _scatter(0)

    @pl.when(wid == _NW - 1)
    def _tail():
        for j in range(_TAIL // 16, _CHUNK // 16):
            idx_v[1, pl.ds(j * 16, 16)] = jnp.full((16,), N_GRAPHS, jnp.int32)
        pltpu.sync_copy(batch_hbm.at[pl.ds(_TAIL_BASE, _TAIL)],
                        idx_v.at[1, pl.ds(0, _TAIL)])
        pltpu.sync_copy(node_hbm.at[pl.ds(_TAIL_BASE, _TAIL)],
                        node_v.at[1, pl.ds(0, _TAIL)])
        pltpu.sync_copy(edge_hbm.at[pl.ds(_TAIL_BASE, _TAIL)],
                        edge_v.at[1, pl.ds(0, _TAIL)])
        _unpack_edges(1, _TAIL)
        pltpu.sync_copy(node_v.at[1], accn_s.at[idx_v.at[1]], add=True)
        pltpu.sync_copy(ec_v, accec_s.at[idx_v.at[1]], add=True)

    plsc.subcore_barrier()

    @pl.when(sid == 0)
    def _flush():
        pltpu.sync_copy(accn_s.at[pl.ds(0, N_GRAPHS)], out_n.at[cid])
        pltpu.sync_copy(accec_s.at[pl.ds(0, N_GRAPHS)], out_ec.at[cid])


def _mlp_body(nsum_ref, ecsum_ref, u_ref, w1n_ref, w1e_ref,
              w1u_ref, b1_ref, w2_ref, b2_ref, out_ref):
    ec = ecsum_ref[0] + ecsum_ref[1]
    cnt = jnp.maximum(ec[:, D_EDGE:D_EDGE + 1], 1.0)
    nbar = (nsum_ref[0] + nsum_ref[1]) / cnt
    ebar = ec[:, :D_EDGE] / cnt
    hp = jnp.float32
    h = (jnp.dot(nbar, w1n_ref[...], preferred_element_type=hp,
                 precision=lax.Precision.HIGHEST)
         + jnp.dot(ebar, w1e_ref[...], preferred_element_type=hp,
                   precision=lax.Precision.HIGHEST)
         + jnp.dot(u_ref[...], w1u_ref[...], preferred_element_type=hp,
                   precision=lax.Precision.HIGHEST)
         + b1_ref[...][None, :])
    h = jnp.maximum(h, 0.0)
    y = jnp.dot(h, w2_ref[...], preferred_element_type=hp,
                precision=lax.Precision.HIGHEST) + b2_ref[...][None, :]
    out_ref[...] = jnp.maximum(y, 0.0)


_mlp_call = pl.pallas_call(
    _mlp_body,
    out_shape=jax.ShapeDtypeStruct((N_GRAPHS, D_GLOBAL), jnp.float32),
)


def kernel(node_attr_prime, edge_out_bar, u, batch, W1, b1, W2, b2):
    batch = batch.astype(jnp.int32)
    zeros = jnp.zeros((N_GRAPHS, D_NODE), jnp.float32)
    ones = jnp.ones((_CHUNK, D_NODE), jnp.float32)
    nsum, ecsum = _segment_sums(node_attr_prime, edge_out_bar, batch,
                                zeros, ones)
    return _mlp_call(nsum, ecsum, u,
                     W1[:D_NODE], W1[D_NODE:D_NODE + D_EDGE],
                     W1[D_NODE + D_EDGE:], b1, W2, b2)
